# baseline (speedup 1.0000x reference)
import functools

import jax
import jax.numpy as jnp
from jax import lax
from jax.experimental import pallas as pl
from jax.experimental.pallas import tpu as pltpu
from jax.experimental.pallas import tpu_sc as plsc

VOCAB = 1_000_000
HIDDEN = 64
BATCH = 4096
HIST = 200
TOTAL = BATCH * HIST

NUM_CORES = 2
NUM_SUBCORES = 16
NW = NUM_CORES * NUM_SUBCORES
PER_W = TOTAL // NW

CHUNK = 128
N_CHUNK = PER_W // CHUNK
GATHERS_PER_ROUND = 4
ROUND = CHUNK * GATHERS_PER_ROUND
N_ROUND = PER_W // ROUND

_mesh = plsc.VectorSubcoreMesh(core_axis_name="c", subcore_axis_name="s")


@functools.partial(
    pl.kernel,
    mesh=_mesh,
    compiler_params=pltpu.CompilerParams(use_tc_tiling_on_sc=False),
    out_type=jax.ShapeDtypeStruct((TOTAL, HIDDEN), jnp.float32),
    scratch_types=[
        pltpu.VMEM((N_CHUNK, CHUNK), jnp.int32),
        pltpu.VMEM((ROUND, HIDDEN), jnp.float32),
        pltpu.SemaphoreType.DMA,
    ],
)
def _emb_lookup(ids_hbm, table_hbm, out_hbm, idx_v, rows_v, gsem):
    wid = lax.axis_index("s") * NUM_CORES + lax.axis_index("c")
    base = wid * PER_W
    pltpu.sync_copy(ids_hbm.at[wid], idx_v)

    def round_body(r, _):
        copies = []
        for g in range(GATHERS_PER_ROUND):
            copies.append(
                pltpu.async_copy(
                    table_hbm.at[idx_v.at[r * GATHERS_PER_ROUND + g]],
                    rows_v.at[pl.ds(g * CHUNK, CHUNK)],
                    gsem,
                )
            )
        for c in copies:
            c.wait()
        pltpu.sync_copy(rows_v, out_hbm.at[pl.ds(base + r * ROUND, ROUND)])
        return ()

    lax.fori_loop(0, N_ROUND, round_body, (), unroll=False)


def kernel(weight, input_ids):
    ids = input_ids.reshape(NW, N_CHUNK, CHUNK).astype(jnp.int32)
    out = _emb_lookup(ids, weight)
    return out.reshape(BATCH, HIST, HIDDEN)


# speedup vs baseline: 1.0228x; 1.0228x over previous
import functools

import jax
import jax.numpy as jnp
from jax import lax
from jax.experimental import pallas as pl
from jax.experimental.pallas import tpu as pltpu
from jax.experimental.pallas import tpu_sc as plsc

VOCAB = 1_000_000
HIDDEN = 64
BATCH = 4096
HIST = 200
TOTAL = BATCH * HIST

NUM_CORES = 2
NUM_SUBCORES = 16
NW = NUM_CORES * NUM_SUBCORES
PER_W = TOTAL // NW

CHUNK = 128
N_CHUNK = PER_W // CHUNK
NBUF = 4
LOOK = 2
N_GROUP = N_CHUNK // NBUF

_mesh = plsc.VectorSubcoreMesh(core_axis_name="c", subcore_axis_name="s")


@functools.partial(
    pl.kernel,
    mesh=_mesh,
    compiler_params=pltpu.CompilerParams(use_tc_tiling_on_sc=False),
    out_type=jax.ShapeDtypeStruct((TOTAL, HIDDEN), jnp.float32),
    scratch_types=[
        pltpu.VMEM((N_CHUNK, CHUNK), jnp.int32),
        pltpu.VMEM((NBUF, CHUNK, HIDDEN), jnp.float32),
        pltpu.SemaphoreType.DMA((NBUF,)),
        pltpu.SemaphoreType.DMA((NBUF,)),
    ],
)
def _emb_lookup(ids_hbm, table_hbm, out_hbm, idx_v, rows_v, gsem, osem):
    wid = lax.axis_index("s") * NUM_CORES + lax.axis_index("c")
    base = wid * PER_W
    pltpu.sync_copy(ids_hbm.at[wid], idx_v)

    def g_copy(j, b):
        return pltpu.make_async_copy(
            table_hbm.at[idx_v.at[j]], rows_v.at[b], gsem.at[b]
        )

    def o_copy(p, b):
        return pltpu.make_async_copy(
            rows_v.at[b], out_hbm.at[pl.ds(base + p * CHUNK, CHUNK)], osem.at[b]
        )

    def slot(g, b, head=False, tail=False):
        p = g * NBUF + b
        f = p + LOOK
        bf = (b + LOOK) % NBUF
        if not tail:
            if not head:
                o_copy(f - NBUF, bf).wait()
            g_copy(f, bf).start()
        g_copy(p, b).wait()
        o_copy(p, b).start()

    for b in range(LOOK):
        g_copy(b, b).start()
    for b in range(NBUF):
        slot(0, b, head=b < NBUF - LOOK)

    def group_body(g, _):
        for b in range(NBUF):
            slot(g, b)
        return ()

    lax.fori_loop(1, N_GROUP - 1, group_body, (), unroll=False)

    for b in range(NBUF):
        slot(N_GROUP - 1, b, tail=b >= NBUF - LOOK)
    for b in range(NBUF):
        o_copy(N_CHUNK - NBUF + b, b).wait()


def kernel(weight, input_ids):
    ids = input_ids.reshape(NW, N_CHUNK, CHUNK).astype(jnp.int32)
    out = _emb_lookup(ids, weight)
    return out.reshape(BATCH, HIST, HIDDEN)
